# Initial kernel scaffold; baseline (speedup 1.0000x reference)
import numpy as np
import jax
import jax.numpy as jnp
from jax import lax
from jax.experimental import pallas as pl
from jax.experimental.pallas import tpu as pltpu

B, H, W = 8, 28, 28
C2, C3 = 128, 256
C = C2 + C3
NQ = B * H * W
QT = 128
NQT = NQ // QT
NC = 16384
CT = 2048
NCT = NC // CT
OUT = 224
KNN = 9

_HIGH = jax.lax.Precision.HIGHEST


def _dot(a, b, dn):
    return lax.dot_general(a, b, dimension_numbers=dn,
                           preferred_element_type=jnp.float32,
                           precision=_HIGH)


def _map_matrix():
    R = np.zeros((OUT, H), dtype=np.float64)
    scale = H / OUT
    for x in range(OUT):
        pos = (x + 0.5) * scale - 0.5
        w = np.maximum(0.0, 1.0 - np.abs(pos - np.arange(H)))
        R[x] = w / w.sum()
    sigma = 4.0
    ks = 2 * int(4.0 * sigma + 0.5) + 1
    r = ks // 2
    t = np.arange(ks, dtype=np.float64) - r
    g = np.exp(-0.5 * (t / sigma) ** 2)
    g = g / g.sum()
    G = np.zeros((OUT, OUT), dtype=np.float64)
    for x in range(OUT):
        lo = max(0, x - r)
        hi = min(OUT, x + r + 1)
        G[x, lo:hi] = g[lo - x + r:hi - x + r]
    return (G @ R).astype(np.float32)


_A_MAP = _map_matrix()


def _pool_body(x2_ref, x3_ref, o2_ref, o3_ref):
    def pool(x, h, w, c):
        zr = jnp.zeros((1, w, c), jnp.float32)
        xp = jnp.concatenate([zr, x, zr], axis=0)
        zc = jnp.zeros((h + 2, 1, c), jnp.float32)
        xp = jnp.concatenate([zc, xp, zc], axis=1)
        acc = jnp.zeros((h, w, c), jnp.float32)
        for i in range(3):
            for j in range(3):
                acc = acc + xp[i:i + h, j:j + w, :]
        return acc * (1.0 / 9.0)

    o2_ref[0] = pool(x2_ref[0], H, W, C2)
    o3_ref[0] = pool(x3_ref[0], H // 2, W // 2, C3)


def _pool(f2t, f3t):
    return pl.pallas_call(
        _pool_body,
        grid=(B,),
        in_specs=[
            pl.BlockSpec((1, H, W, C2), lambda b: (b, 0, 0, 0)),
            pl.BlockSpec((1, H // 2, W // 2, C3), lambda b: (b, 0, 0, 0)),
        ],
        out_specs=[
            pl.BlockSpec((1, H, W, C2), lambda b: (b, 0, 0, 0)),
            pl.BlockSpec((1, H // 2, W // 2, C3), lambda b: (b, 0, 0, 0)),
        ],
        out_shape=[
            jax.ShapeDtypeStruct((B, H, W, C2), jnp.float32),
            jax.ShapeDtypeStruct((B, H // 2, W // 2, C3), jnp.float32),
        ],
    )(f2t, f3t)


def _knn_body(emb_ref, cs_ref, score_ref, loc_ref, minv, mini):
    c = pl.program_id(0)
    chunk = cs_ref[...]
    b2 = jnp.sum(chunk * chunk, axis=1)

    def step(q, _):
        eq = emb_ref[pl.ds(q * QT, QT), :]
        a2 = jnp.sum(eq * eq, axis=1)
        ab = _dot(eq, chunk, (((1,), (1,)), ((), ())))
        d2 = a2[:, None] + b2[None, :] - 2.0 * ab
        m = jnp.min(d2, axis=1)
        am = jnp.argmin(d2, axis=1).astype(jnp.int32) + c * CT
        prev_v = jnp.where(c == 0, jnp.inf, minv[q, 0, :])
        prev_i = jnp.where(c == 0, 0, mini[q, 0, :])
        upd = m < prev_v
        new_v = jnp.where(upd, m, prev_v)
        new_i = jnp.where(upd, am, prev_i)
        minv[q, 0, :] = new_v
        mini[q, 0, :] = new_i

        @pl.when(c == NCT - 1)
        def _():
            score_ref[q, 0, :] = jnp.sqrt(jnp.maximum(new_v, 1e-12))
            loc_ref[q, 0, :] = new_i
        return 0

    lax.fori_loop(0, NQT, step, 0, unroll=False)


def _knn(emb, coreset):
    return pl.pallas_call(
        _knn_body,
        grid=(NCT,),
        in_specs=[
            pl.BlockSpec((NQ, C), lambda c: (0, 0)),
            pl.BlockSpec((CT, C), lambda c: (c, 0)),
        ],
        out_specs=[
            pl.BlockSpec((NQT, 1, QT), lambda c: (0, 0, 0)),
            pl.BlockSpec((NQT, 1, QT), lambda c: (0, 0, 0)),
        ],
        out_shape=[
            jax.ShapeDtypeStruct((NQT, 1, QT), jnp.float32),
            jax.ShapeDtypeStruct((NQT, 1, QT), jnp.int32),
        ],
        scratch_shapes=[
            pltpu.VMEM((NQT, 1, QT), jnp.float32),
            pltpu.VMEM((NQT, 1, QT), jnp.int32),
        ],
    )(emb, coreset)


def _score_body(ps_ref, loc_ref, emb_ref, cs_ref, out_ref, nns_ref, dnn_ref,
                dmp_ref):
    i = pl.program_id(0)
    s = i // NCT
    cidx = i % NCT
    chunk = cs_ref[...]

    ps = ps_ref[...]
    locs = loc_ref[...]
    mp = jnp.argmax(ps, axis=1).astype(jnp.int32)
    iota_p = lax.broadcasted_iota(jnp.int32, (B, H * W), 1)
    sel = iota_p == mp[:, None]
    nn_index = jnp.sum(jnp.where(sel, locs, 0), axis=1)
    iota_c = lax.broadcasted_iota(jnp.int32, (B, NC), 1)
    nn_onehot = (iota_c == nn_index[:, None]).astype(jnp.float32)

    @pl.when(s == 0)
    def _():
        oh_c = nn_onehot[:, pl.ds(cidx * CT, CT)]
        contrib = _dot(oh_c, chunk, (((1,), (0,)), ((), ())))
        prev = jnp.where(i == 0, 0.0, nns_ref[...])
        nns_ref[...] = prev + contrib

    @pl.when(s == 1)
    def _():
        row = lax.broadcasted_iota(jnp.int32, (B, 1), 0)[:, 0] * (H * W) + mp
        iota_e = lax.broadcasted_iota(jnp.int32, (B, NQ), 1)
        oh_e = (iota_e == row[:, None]).astype(jnp.float32)
        mpf = _dot(oh_e, emb_ref[...], (((1,), (0,)), ((), ())))
        nns = nns_ref[...]
        b2 = jnp.sum(chunk * chunk, axis=1)
        n2 = jnp.sum(nns * nns, axis=1)
        m2 = jnp.sum(mpf * mpf, axis=1)
        dn = n2[:, None] + b2[None, :] - 2.0 * _dot(
            nns, chunk, (((1,), (1,)), ((), ())))
        dm = m2[:, None] + b2[None, :] - 2.0 * _dot(
            mpf, chunk, (((1,), (1,)), ((), ())))
        dnn_ref[:, pl.ds(cidx * CT, CT)] = jnp.sqrt(jnp.maximum(dn, 1e-12))
        dmp_ref[:, pl.ds(cidx * CT, CT)] = jnp.sqrt(jnp.maximum(dm, 1e-12))

    @pl.when(i == 2 * NCT - 1)
    def _():
        dd = dnn_ref[...]
        dmp = dmp_ref[...]
        dsup = []
        for _k in range(KNN):
            am = jnp.argmin(dd, axis=1).astype(jnp.int32)
            mask = iota_c == am[:, None]
            dsup.append(jnp.sum(jnp.where(mask, dmp, 0.0), axis=1))
            dd = jnp.where(mask, jnp.inf, dd)
        dsup = jnp.stack(dsup, axis=1)
        mx = jnp.max(dsup, axis=1, keepdims=True)
        e = jnp.exp(dsup - mx)
        wgt = 1.0 - e[:, 0] / jnp.sum(e, axis=1)
        score = jnp.max(ps, axis=1)
        out_ref[...] = (wgt * score)[:, None]


def _score(ps, locs, emb, coreset):
    return pl.pallas_call(
        _score_body,
        grid=(2 * NCT,),
        in_specs=[
            pl.BlockSpec((B, H * W), lambda i: (0, 0)),
            pl.BlockSpec((B, H * W), lambda i: (0, 0)),
            pl.BlockSpec((NQ, C), lambda i: (0, 0)),
            pl.BlockSpec((CT, C), lambda i: (i % NCT, 0)),
        ],
        out_specs=pl.BlockSpec((B, 1), lambda i: (0, 0)),
        out_shape=jax.ShapeDtypeStruct((B, 1), jnp.float32),
        scratch_shapes=[
            pltpu.VMEM((B, C), jnp.float32),
            pltpu.VMEM((B, NC), jnp.float32),
            pltpu.VMEM((B, NC), jnp.float32),
        ],
    )(ps, locs, emb, coreset)


def _amap_body(ps_ref, a_ref, o_ref):
    a = a_ref[...]
    p = ps_ref[0]
    t = _dot(a, p, (((1,), (0,)), ((), ())))
    o_ref[0] = _dot(t, a, (((1,), (1,)), ((), ())))


def _amap(ps_img, a_mat):
    return pl.pallas_call(
        _amap_body,
        grid=(B,),
        in_specs=[
            pl.BlockSpec((1, H, W), lambda b: (b, 0, 0)),
            pl.BlockSpec((OUT, H), lambda b: (0, 0)),
        ],
        out_specs=pl.BlockSpec((1, OUT, OUT), lambda b: (b, 0, 0)),
        out_shape=jax.ShapeDtypeStruct((B, OUT, OUT), jnp.float32),
    )(ps_img, a_mat)


def kernel(feat_layer2, feat_layer3, embedding_coreset):
    f2t = jnp.transpose(feat_layer2, (0, 2, 3, 1))
    f3t = jnp.transpose(feat_layer3, (0, 2, 3, 1))
    p2, p3 = _pool(f2t, f3t)
    up3 = jnp.broadcast_to(
        p3[:, :, None, :, None, :],
        (B, H // 2, 2, W // 2, 2, C3)).reshape(B, H, W, C3)
    emb = jnp.concatenate(
        [p2.reshape(NQ, C2), up3.reshape(NQ, C3)], axis=1)
    scores, locs = _knn(emb, embedding_coreset)
    ps = scores.reshape(B, H * W)
    lc = locs.reshape(B, H * W)
    a_score = _score(ps, lc, emb, embedding_coreset).reshape(B)
    amap = _amap(ps.reshape(B, H, W), jnp.asarray(_A_MAP))
    return amap.reshape(B, 1, OUT, OUT), a_score


# speedup vs baseline: 3.0791x; 3.0791x over previous
import numpy as np
import jax
import jax.numpy as jnp
from jax import lax
from jax.experimental import pallas as pl
from jax.experimental.pallas import tpu as pltpu

B, H, W = 8, 28, 28
C2, C3 = 128, 256
C = C2 + C3
NQ = B * H * W
QT = 128
NQT = NQ // QT
NC = 16384
CT = 2048
NCT = NC // CT
OUT = 224
KNN = 9

_HIGH = jax.lax.Precision.HIGHEST


def _dot(a, b, dn):
    return lax.dot_general(a, b, dimension_numbers=dn,
                           preferred_element_type=jnp.float32,
                           precision=_HIGH)


def _map_matrix():
    R = np.zeros((OUT, H), dtype=np.float64)
    scale = H / OUT
    for x in range(OUT):
        pos = (x + 0.5) * scale - 0.5
        w = np.maximum(0.0, 1.0 - np.abs(pos - np.arange(H)))
        R[x] = w / w.sum()
    sigma = 4.0
    ks = 2 * int(4.0 * sigma + 0.5) + 1
    r = ks // 2
    t = np.arange(ks, dtype=np.float64) - r
    g = np.exp(-0.5 * (t / sigma) ** 2)
    g = g / g.sum()
    G = np.zeros((OUT, OUT), dtype=np.float64)
    for x in range(OUT):
        lo = max(0, x - r)
        hi = min(OUT, x + r + 1)
        G[x, lo:hi] = g[lo - x + r:hi - x + r]
    return (G @ R).astype(np.float32)


_A_MAP = _map_matrix()


def _pool_body(x2_ref, x3_ref, o2_ref, o3_ref):
    def pool(x, h, w, c):
        zr = jnp.zeros((1, w, c), jnp.float32)
        xp = jnp.concatenate([zr, x, zr], axis=0)
        zc = jnp.zeros((h + 2, 1, c), jnp.float32)
        xp = jnp.concatenate([zc, xp, zc], axis=1)
        acc = jnp.zeros((h, w, c), jnp.float32)
        for i in range(3):
            for j in range(3):
                acc = acc + xp[i:i + h, j:j + w, :]
        return acc * (1.0 / 9.0)

    o2_ref[0] = pool(x2_ref[0], H, W, C2)
    o3_ref[0] = pool(x3_ref[0], H // 2, W // 2, C3)


def _pool(f2t, f3t):
    return pl.pallas_call(
        _pool_body,
        grid=(B,),
        in_specs=[
            pl.BlockSpec((1, H, W, C2), lambda b: (b, 0, 0, 0)),
            pl.BlockSpec((1, H // 2, W // 2, C3), lambda b: (b, 0, 0, 0)),
        ],
        out_specs=[
            pl.BlockSpec((1, H, W, C2), lambda b: (b, 0, 0, 0)),
            pl.BlockSpec((1, H // 2, W // 2, C3), lambda b: (b, 0, 0, 0)),
        ],
        out_shape=[
            jax.ShapeDtypeStruct((B, H, W, C2), jnp.float32),
            jax.ShapeDtypeStruct((B, H // 2, W // 2, C3), jnp.float32),
        ],
    )(f2t, f3t)


def _knn_body(emb_ref, cs_ref, score_ref, loc_ref, minv, mini):
    c = pl.program_id(0)
    chunk = cs_ref[...]
    b2 = jnp.sum(chunk * chunk, axis=1)

    def step(q, _):
        eq = emb_ref[pl.ds(q * QT, QT), :]
        a2 = jnp.sum(eq * eq, axis=1)
        ab = _dot(eq, chunk, (((1,), (1,)), ((), ())))
        d2 = a2[:, None] + b2[None, :] - 2.0 * ab
        m = jnp.min(d2, axis=1)
        am = jnp.argmin(d2, axis=1).astype(jnp.int32) + c * CT
        prev_v = jnp.where(c == 0, jnp.inf, minv[q, 0, :])
        prev_i = jnp.where(c == 0, 0, mini[q, 0, :])
        upd = m < prev_v
        new_v = jnp.where(upd, m, prev_v)
        new_i = jnp.where(upd, am, prev_i)
        minv[q, 0, :] = new_v
        mini[q, 0, :] = new_i

        @pl.when(c == NCT - 1)
        def _():
            score_ref[q, 0, :] = jnp.sqrt(jnp.maximum(new_v, 1e-12))
            loc_ref[q, 0, :] = new_i
        return 0

    lax.fori_loop(0, NQT, step, 0, unroll=False)


def _knn(emb, coreset):
    return pl.pallas_call(
        _knn_body,
        grid=(NCT,),
        in_specs=[
            pl.BlockSpec((NQ, C), lambda c: (0, 0)),
            pl.BlockSpec((CT, C), lambda c: (c, 0)),
        ],
        out_specs=[
            pl.BlockSpec((NQT, 1, QT), lambda c: (0, 0, 0)),
            pl.BlockSpec((NQT, 1, QT), lambda c: (0, 0, 0)),
        ],
        out_shape=[
            jax.ShapeDtypeStruct((NQT, 1, QT), jnp.float32),
            jax.ShapeDtypeStruct((NQT, 1, QT), jnp.int32),
        ],
        scratch_shapes=[
            pltpu.VMEM((NQT, 1, QT), jnp.float32),
            pltpu.VMEM((NQT, 1, QT), jnp.int32),
        ],
    )(emb, coreset)


def _score_body(ps_ref, loc_ref, emb_ref, cs_ref, out_ref, nns_ref, dnn_ref,
                dmp_ref):
    i = pl.program_id(0)
    s = i // NCT
    cidx = i % NCT
    chunk = cs_ref[...]

    ps = ps_ref[...]
    locs = loc_ref[...]
    mp = jnp.argmax(ps, axis=1).astype(jnp.int32)
    iota_p = lax.broadcasted_iota(jnp.int32, (B, H * W), 1)
    sel = iota_p == mp[:, None]
    nn_index = jnp.sum(jnp.where(sel, locs, 0), axis=1)
    iota_c = lax.broadcasted_iota(jnp.int32, (B, NC), 1)

    @pl.when(s == 0)
    def _():
        iota_ct = lax.broadcasted_iota(jnp.int32, (B, CT), 1) + cidx * CT
        oh_c = (iota_ct == nn_index[:, None]).astype(jnp.float32)
        contrib = _dot(oh_c, chunk, (((1,), (0,)), ((), ())))
        prev = jnp.where(i == 0, 0.0, nns_ref[...])
        nns_ref[...] = prev + contrib

    @pl.when(s == 1)
    def _():
        row = lax.broadcasted_iota(jnp.int32, (B, 1), 0)[:, 0] * (H * W) + mp
        iota_e = lax.broadcasted_iota(jnp.int32, (B, NQ), 1)
        oh_e = (iota_e == row[:, None]).astype(jnp.float32)
        mpf = _dot(oh_e, emb_ref[...], (((1,), (0,)), ((), ())))
        nns = nns_ref[...]
        b2 = jnp.sum(chunk * chunk, axis=1)
        n2 = jnp.sum(nns * nns, axis=1)
        m2 = jnp.sum(mpf * mpf, axis=1)
        dn = n2[:, None] + b2[None, :] - 2.0 * _dot(
            nns, chunk, (((1,), (1,)), ((), ())))
        dm = m2[:, None] + b2[None, :] - 2.0 * _dot(
            mpf, chunk, (((1,), (1,)), ((), ())))
        dnn_ref[:, pl.ds(cidx * CT, CT)] = jnp.sqrt(jnp.maximum(dn, 1e-12))
        dmp_ref[:, pl.ds(cidx * CT, CT)] = jnp.sqrt(jnp.maximum(dm, 1e-12))

    @pl.when(i == 2 * NCT - 1)
    def _():
        dd = dnn_ref[...]
        dmp = dmp_ref[...]
        dsup = []
        for _k in range(KNN):
            am = jnp.argmin(dd, axis=1).astype(jnp.int32)
            mask = iota_c == am[:, None]
            dsup.append(jnp.sum(jnp.where(mask, dmp, 0.0), axis=1))
            dd = jnp.where(mask, jnp.inf, dd)
        dsup = jnp.stack(dsup, axis=1)
        mx = jnp.max(dsup, axis=1, keepdims=True)
        e = jnp.exp(dsup - mx)
        wgt = 1.0 - e[:, 0] / jnp.sum(e, axis=1)
        score = jnp.max(ps, axis=1)
        out_ref[...] = (wgt * score)[:, None]


def _score(ps, locs, emb, coreset):
    return pl.pallas_call(
        _score_body,
        grid=(2 * NCT,),
        in_specs=[
            pl.BlockSpec((B, H * W), lambda i: (0, 0)),
            pl.BlockSpec((B, H * W), lambda i: (0, 0)),
            pl.BlockSpec((NQ, C), lambda i: (0, 0)),
            pl.BlockSpec((CT, C), lambda i: (i % NCT, 0)),
        ],
        out_specs=pl.BlockSpec((B, 1), lambda i: (0, 0)),
        out_shape=jax.ShapeDtypeStruct((B, 1), jnp.float32),
        scratch_shapes=[
            pltpu.VMEM((B, C), jnp.float32),
            pltpu.VMEM((B, NC), jnp.float32),
            pltpu.VMEM((B, NC), jnp.float32),
        ],
    )(ps, locs, emb, coreset)


def _amap_body(ps_ref, a_ref, o_ref):
    a = a_ref[...]
    p = ps_ref[0]
    t = _dot(a, p, (((1,), (0,)), ((), ())))
    o_ref[0] = _dot(t, a, (((1,), (1,)), ((), ())))


def _amap(ps_img, a_mat):
    return pl.pallas_call(
        _amap_body,
        grid=(B,),
        in_specs=[
            pl.BlockSpec((1, H, W), lambda b: (b, 0, 0)),
            pl.BlockSpec((OUT, H), lambda b: (0, 0)),
        ],
        out_specs=pl.BlockSpec((1, OUT, OUT), lambda b: (b, 0, 0)),
        out_shape=jax.ShapeDtypeStruct((B, OUT, OUT), jnp.float32),
    )(ps_img, a_mat)


def kernel(feat_layer2, feat_layer3, embedding_coreset):
    f2t = jnp.transpose(feat_layer2, (0, 2, 3, 1))
    f3t = jnp.transpose(feat_layer3, (0, 2, 3, 1))
    p2, p3 = _pool(f2t, f3t)
    up3 = jnp.broadcast_to(
        p3[:, :, None, :, None, :],
        (B, H // 2, 2, W // 2, 2, C3)).reshape(B, H, W, C3)
    emb = jnp.concatenate(
        [p2.reshape(NQ, C2), up3.reshape(NQ, C3)], axis=1)
    scores, locs = _knn(emb, embedding_coreset)
    ps = scores.reshape(B, H * W)
    lc = locs.reshape(B, H * W)
    a_score = _score(ps, lc, emb, embedding_coreset).reshape(B)
    amap = _amap(ps.reshape(B, H, W), jnp.asarray(_A_MAP))
    return amap.reshape(B, 1, OUT, OUT), a_score


# speedup vs baseline: 4.8335x; 1.5698x over previous
import numpy as np
import jax
import jax.numpy as jnp
from jax import lax
from jax.experimental import pallas as pl
from jax.experimental.pallas import tpu as pltpu

B, H, W = 8, 28, 28
C2, C3 = 128, 256
C = C2 + C3
NQ = B * H * W
QT = 128
NQT = NQ // QT
NC = 16384
CT = 2048
NCT = NC // CT
OUT = 224
KNN = 9

_HIGH = jax.lax.Precision.HIGHEST


def _dot(a, b, dn, prec=_HIGH):
    return lax.dot_general(a, b, dimension_numbers=dn,
                           preferred_element_type=jnp.float32,
                           precision=prec)


_DN_T = (((1,), (1,)), ((), ()))


def _dot3(a, b):
    a_hi = a.astype(jnp.bfloat16)
    a_lo = (a - a_hi.astype(jnp.float32)).astype(jnp.bfloat16)
    b_hi = b.astype(jnp.bfloat16)
    b_lo = (b - b_hi.astype(jnp.float32)).astype(jnp.bfloat16)
    d = jax.lax.Precision.DEFAULT
    hi = _dot(a_hi, b_hi, _DN_T, prec=d)
    m1 = _dot(a_hi, b_lo, _DN_T, prec=d)
    m2 = _dot(a_lo, b_hi, _DN_T, prec=d)
    return hi + m1 + m2


def _map_matrix():
    R = np.zeros((OUT, H), dtype=np.float64)
    scale = H / OUT
    for x in range(OUT):
        pos = (x + 0.5) * scale - 0.5
        w = np.maximum(0.0, 1.0 - np.abs(pos - np.arange(H)))
        R[x] = w / w.sum()
    sigma = 4.0
    ks = 2 * int(4.0 * sigma + 0.5) + 1
    r = ks // 2
    t = np.arange(ks, dtype=np.float64) - r
    g = np.exp(-0.5 * (t / sigma) ** 2)
    g = g / g.sum()
    G = np.zeros((OUT, OUT), dtype=np.float64)
    for x in range(OUT):
        lo = max(0, x - r)
        hi = min(OUT, x + r + 1)
        G[x, lo:hi] = g[lo - x + r:hi - x + r]
    return (G @ R).astype(np.float32)


_A_MAP = _map_matrix()


def _pool_body(x2_ref, x3_ref, o2_ref, o3_ref):
    def pool(x, h, w, c):
        zr = jnp.zeros((1, w, c), jnp.float32)
        xp = jnp.concatenate([zr, x, zr], axis=0)
        zc = jnp.zeros((h + 2, 1, c), jnp.float32)
        xp = jnp.concatenate([zc, xp, zc], axis=1)
        acc = jnp.zeros((h, w, c), jnp.float32)
        for i in range(3):
            for j in range(3):
                acc = acc + xp[i:i + h, j:j + w, :]
        return acc * (1.0 / 9.0)

    o2_ref[0] = pool(x2_ref[0], H, W, C2)
    o3_ref[0] = pool(x3_ref[0], H // 2, W // 2, C3)


def _pool(f2t, f3t):
    return pl.pallas_call(
        _pool_body,
        grid=(B,),
        in_specs=[
            pl.BlockSpec((1, H, W, C2), lambda b: (b, 0, 0, 0)),
            pl.BlockSpec((1, H // 2, W // 2, C3), lambda b: (b, 0, 0, 0)),
        ],
        out_specs=[
            pl.BlockSpec((1, H, W, C2), lambda b: (b, 0, 0, 0)),
            pl.BlockSpec((1, H // 2, W // 2, C3), lambda b: (b, 0, 0, 0)),
        ],
        out_shape=[
            jax.ShapeDtypeStruct((B, H, W, C2), jnp.float32),
            jax.ShapeDtypeStruct((B, H // 2, W // 2, C3), jnp.float32),
        ],
    )(f2t, f3t)


def _knn_body(emb_ref, cs_ref, score_ref, loc_ref, minv, mini):
    c = pl.program_id(0)
    chunk = cs_ref[...]
    b2 = jnp.sum(chunk * chunk, axis=1)

    def step(q, _):
        eq = emb_ref[pl.ds(q * QT, QT), :]
        a2 = jnp.sum(eq * eq, axis=1)
        ab = _dot3(eq, chunk)
        d2 = a2[:, None] + b2[None, :] - 2.0 * ab
        m = jnp.min(d2, axis=1)
        am = jnp.argmin(d2, axis=1).astype(jnp.int32) + c * CT
        prev_v = jnp.where(c == 0, jnp.inf, minv[q, 0, :])
        prev_i = jnp.where(c == 0, 0, mini[q, 0, :])
        upd = m < prev_v
        new_v = jnp.where(upd, m, prev_v)
        new_i = jnp.where(upd, am, prev_i)
        minv[q, 0, :] = new_v
        mini[q, 0, :] = new_i

        @pl.when(c == NCT - 1)
        def _():
            score_ref[q, 0, :] = jnp.sqrt(jnp.maximum(new_v, 1e-12))
            loc_ref[q, 0, :] = new_i
        return 0

    lax.fori_loop(0, NQT, step, 0, unroll=False)


def _knn(emb, coreset):
    return pl.pallas_call(
        _knn_body,
        grid=(NCT,),
        in_specs=[
            pl.BlockSpec((NQ, C), lambda c: (0, 0)),
            pl.BlockSpec((CT, C), lambda c: (c, 0)),
        ],
        out_specs=[
            pl.BlockSpec((NQT, 1, QT), lambda c: (0, 0, 0)),
            pl.BlockSpec((NQT, 1, QT), lambda c: (0, 0, 0)),
        ],
        out_shape=[
            jax.ShapeDtypeStruct((NQT, 1, QT), jnp.float32),
            jax.ShapeDtypeStruct((NQT, 1, QT), jnp.int32),
        ],
        scratch_shapes=[
            pltpu.VMEM((NQT, 1, QT), jnp.float32),
            pltpu.VMEM((NQT, 1, QT), jnp.int32),
        ],
    )(emb, coreset)


def _score_body(ps_ref, loc_ref, emb_ref, cs_ref, out_ref, nns_ref, dnn_ref,
                dmp_ref):
    i = pl.program_id(0)
    s = i // NCT
    cidx = i % NCT
    chunk = cs_ref[...]

    ps = ps_ref[...]
    locs = loc_ref[...]
    mp = jnp.argmax(ps, axis=1).astype(jnp.int32)
    iota_p = lax.broadcasted_iota(jnp.int32, (B, H * W), 1)
    sel = iota_p == mp[:, None]
    nn_index = jnp.sum(jnp.where(sel, locs, 0), axis=1)
    iota_c = lax.broadcasted_iota(jnp.int32, (B, NC), 1)

    @pl.when(s == 0)
    def _():
        iota_ct = lax.broadcasted_iota(jnp.int32, (B, CT), 1) + cidx * CT
        oh_c = (iota_ct == nn_index[:, None]).astype(jnp.float32)
        contrib = _dot(oh_c, chunk, (((1,), (0,)), ((), ())))
        prev = jnp.where(i == 0, 0.0, nns_ref[...])
        nns_ref[...] = prev + contrib

    @pl.when(s == 1)
    def _():
        row = lax.broadcasted_iota(jnp.int32, (B, 1), 0)[:, 0] * (H * W) + mp
        iota_e = lax.broadcasted_iota(jnp.int32, (B, NQ), 1)
        oh_e = (iota_e == row[:, None]).astype(jnp.float32)
        mpf = _dot(oh_e, emb_ref[...], (((1,), (0,)), ((), ())))
        nns = nns_ref[...]
        b2 = jnp.sum(chunk * chunk, axis=1)
        n2 = jnp.sum(nns * nns, axis=1)
        m2 = jnp.sum(mpf * mpf, axis=1)
        dn = n2[:, None] + b2[None, :] - 2.0 * _dot(
            nns, chunk, (((1,), (1,)), ((), ())))
        dm = m2[:, None] + b2[None, :] - 2.0 * _dot(
            mpf, chunk, (((1,), (1,)), ((), ())))
        dnn_ref[:, pl.ds(cidx * CT, CT)] = jnp.sqrt(jnp.maximum(dn, 1e-12))
        dmp_ref[:, pl.ds(cidx * CT, CT)] = jnp.sqrt(jnp.maximum(dm, 1e-12))

    @pl.when(i == 2 * NCT - 1)
    def _():
        dd = dnn_ref[...]
        dmp = dmp_ref[...]
        dsup = []
        for _k in range(KNN):
            am = jnp.argmin(dd, axis=1).astype(jnp.int32)
            mask = iota_c == am[:, None]
            dsup.append(jnp.sum(jnp.where(mask, dmp, 0.0), axis=1))
            dd = jnp.where(mask, jnp.inf, dd)
        dsup = jnp.stack(dsup, axis=1)
        mx = jnp.max(dsup, axis=1, keepdims=True)
        e = jnp.exp(dsup - mx)
        wgt = 1.0 - e[:, 0] / jnp.sum(e, axis=1)
        score = jnp.max(ps, axis=1)
        out_ref[...] = (wgt * score)[:, None]


def _score(ps, locs, emb, coreset):
    return pl.pallas_call(
        _score_body,
        grid=(2 * NCT,),
        in_specs=[
            pl.BlockSpec((B, H * W), lambda i: (0, 0)),
            pl.BlockSpec((B, H * W), lambda i: (0, 0)),
            pl.BlockSpec((NQ, C), lambda i: (0, 0)),
            pl.BlockSpec((CT, C), lambda i: (i % NCT, 0)),
        ],
        out_specs=pl.BlockSpec((B, 1), lambda i: (0, 0)),
        out_shape=jax.ShapeDtypeStruct((B, 1), jnp.float32),
        scratch_shapes=[
            pltpu.VMEM((B, C), jnp.float32),
            pltpu.VMEM((B, NC), jnp.float32),
            pltpu.VMEM((B, NC), jnp.float32),
        ],
    )(ps, locs, emb, coreset)


def _amap_body(ps_ref, a_ref, o_ref):
    a = a_ref[...]
    p = ps_ref[0]
    t = _dot(a, p, (((1,), (0,)), ((), ())))
    o_ref[0] = _dot(t, a, (((1,), (1,)), ((), ())))


def _amap(ps_img, a_mat):
    return pl.pallas_call(
        _amap_body,
        grid=(B,),
        in_specs=[
            pl.BlockSpec((1, H, W), lambda b: (b, 0, 0)),
            pl.BlockSpec((OUT, H), lambda b: (0, 0)),
        ],
        out_specs=pl.BlockSpec((1, OUT, OUT), lambda b: (b, 0, 0)),
        out_shape=jax.ShapeDtypeStruct((B, OUT, OUT), jnp.float32),
    )(ps_img, a_mat)


def kernel(feat_layer2, feat_layer3, embedding_coreset):
    f2t = jnp.transpose(feat_layer2, (0, 2, 3, 1))
    f3t = jnp.transpose(feat_layer3, (0, 2, 3, 1))
    p2, p3 = _pool(f2t, f3t)
    up3 = jnp.broadcast_to(
        p3[:, :, None, :, None, :],
        (B, H // 2, 2, W // 2, 2, C3)).reshape(B, H, W, C3)
    emb = jnp.concatenate(
        [p2.reshape(NQ, C2), up3.reshape(NQ, C3)], axis=1)
    scores, locs = _knn(emb, embedding_coreset)
    ps = scores.reshape(B, H * W)
    lc = locs.reshape(B, H * W)
    a_score = _score(ps, lc, emb, embedding_coreset).reshape(B)
    amap = _amap(ps.reshape(B, H, W), jnp.asarray(_A_MAP))
    return amap.reshape(B, 1, OUT, OUT), a_score


# speedup vs baseline: 5.0551x; 1.0459x over previous
import numpy as np
import jax
import jax.numpy as jnp
from jax import lax
from jax.experimental import pallas as pl
from jax.experimental.pallas import tpu as pltpu

B, H, W = 8, 28, 28
C2, C3 = 128, 256
C = C2 + C3
NQ = B * H * W
QT = 128
NQT = NQ // QT
NC = 16384
CT = 2048
NCT = NC // CT
OUT = 224
KNN = 9

_HIGH = jax.lax.Precision.HIGHEST


def _dot(a, b, dn, prec=_HIGH):
    return lax.dot_general(a, b, dimension_numbers=dn,
                           preferred_element_type=jnp.float32,
                           precision=prec)


_DN_T = (((1,), (1,)), ((), ()))


def _dot3(a, b):
    a_hi = a.astype(jnp.bfloat16)
    a_lo = (a - a_hi.astype(jnp.float32)).astype(jnp.bfloat16)
    b_hi = b.astype(jnp.bfloat16)
    b_lo = (b - b_hi.astype(jnp.float32)).astype(jnp.bfloat16)
    d = jax.lax.Precision.DEFAULT
    hi = _dot(a_hi, b_hi, _DN_T, prec=d)
    m1 = _dot(a_hi, b_lo, _DN_T, prec=d)
    m2 = _dot(a_lo, b_hi, _DN_T, prec=d)
    return hi + m1 + m2


def _map_matrix():
    R = np.zeros((OUT, H), dtype=np.float64)
    scale = H / OUT
    for x in range(OUT):
        pos = (x + 0.5) * scale - 0.5
        w = np.maximum(0.0, 1.0 - np.abs(pos - np.arange(H)))
        R[x] = w / w.sum()
    sigma = 4.0
    ks = 2 * int(4.0 * sigma + 0.5) + 1
    r = ks // 2
    t = np.arange(ks, dtype=np.float64) - r
    g = np.exp(-0.5 * (t / sigma) ** 2)
    g = g / g.sum()
    G = np.zeros((OUT, OUT), dtype=np.float64)
    for x in range(OUT):
        lo = max(0, x - r)
        hi = min(OUT, x + r + 1)
        G[x, lo:hi] = g[lo - x + r:hi - x + r]
    return (G @ R).astype(np.float32)


_A_MAP = _map_matrix()


def _pool_body(x2_ref, x3_ref, o2_ref, o3_ref):
    def pool(x, h, w, c):
        zr = jnp.zeros((1, w, c), jnp.float32)
        xp = jnp.concatenate([zr, x, zr], axis=0)
        zc = jnp.zeros((h + 2, 1, c), jnp.float32)
        xp = jnp.concatenate([zc, xp, zc], axis=1)
        acc = jnp.zeros((h, w, c), jnp.float32)
        for i in range(3):
            for j in range(3):
                acc = acc + xp[i:i + h, j:j + w, :]
        return acc * (1.0 / 9.0)

    o2_ref[0] = pool(x2_ref[0], H, W, C2)
    o3_ref[0] = pool(x3_ref[0], H // 2, W // 2, C3)


def _pool(f2t, f3t):
    return pl.pallas_call(
        _pool_body,
        grid=(B,),
        in_specs=[
            pl.BlockSpec((1, H, W, C2), lambda b: (b, 0, 0, 0)),
            pl.BlockSpec((1, H // 2, W // 2, C3), lambda b: (b, 0, 0, 0)),
        ],
        out_specs=[
            pl.BlockSpec((1, H, W, C2), lambda b: (b, 0, 0, 0)),
            pl.BlockSpec((1, H // 2, W // 2, C3), lambda b: (b, 0, 0, 0)),
        ],
        out_shape=[
            jax.ShapeDtypeStruct((B, H, W, C2), jnp.float32),
            jax.ShapeDtypeStruct((B, H // 2, W // 2, C3), jnp.float32),
        ],
    )(f2t, f3t)


def _knn_body(emb_ref, cs_ref, score_ref, minv, chi, clo, b2s):
    c = pl.program_id(0)
    q = pl.program_id(1)

    @pl.when(q == 0)
    def _():
        ch = cs_ref[...]
        b2s[...] = jnp.broadcast_to(
            jnp.sum(ch * ch, axis=1)[None, :], (8, CT))
        cn = ch * (-2.0)
        hi = cn.astype(jnp.bfloat16)
        chi[...] = hi
        clo[...] = (cn - hi.astype(jnp.float32)).astype(jnp.bfloat16)

    dflt = jax.lax.Precision.DEFAULT
    eq = emb_ref[...]
    a2 = jnp.sum(eq * eq, axis=1)
    e_hi = eq.astype(jnp.bfloat16)
    e_lo = (eq - e_hi.astype(jnp.float32)).astype(jnp.bfloat16)
    c_hi = chi[...]
    c_lo = clo[...]
    ab = (_dot(e_hi, c_hi, _DN_T, prec=dflt)
          + _dot(e_hi, c_lo, _DN_T, prec=dflt)
          + _dot(e_lo, c_hi, _DN_T, prec=dflt))
    m = jnp.min(b2s[0:1, :] + ab, axis=1)
    prev_v = jnp.where(c == 0, jnp.inf, minv[q, 0, :])
    new_v = jnp.minimum(m, prev_v)

    @pl.when(c == NCT - 1)
    def _():
        score_ref[0, 0, :] = jnp.sqrt(jnp.maximum(a2 + new_v, 1e-12))

    @pl.when(c < NCT - 1)
    def _():
        minv[q, 0, :] = new_v


def _knn(emb, coreset):
    return pl.pallas_call(
        _knn_body,
        grid=(NCT, NQT),
        in_specs=[
            pl.BlockSpec((QT, C), lambda c, q: (q, 0)),
            pl.BlockSpec((CT, C), lambda c, q: (c, 0)),
        ],
        out_specs=pl.BlockSpec((1, 1, QT), lambda c, q: (q, 0, 0)),
        out_shape=jax.ShapeDtypeStruct((NQT, 1, QT), jnp.float32),
        scratch_shapes=[
            pltpu.VMEM((NQT, 1, QT), jnp.float32),
            pltpu.VMEM((CT, C), jnp.bfloat16),
            pltpu.VMEM((CT, C), jnp.bfloat16),
            pltpu.VMEM((8, CT), jnp.float32),
        ],
    )(emb, coreset)


def _score_body(ps_ref, emb_ref, cs_ref, out_ref, mpf_s, bsc_s, nns_ref,
                runm_ref, dnn_ref, dmp_ref):
    i = pl.program_id(0)
    s = i // NCT
    cidx = i % NCT
    chunk = cs_ref[...]
    b2 = jnp.sum(chunk * chunk, axis=1)

    @pl.when(i == 0)
    def _():
        ps = ps_ref[...]
        mx = jnp.max(ps, axis=1)
        iota_p = lax.broadcasted_iota(jnp.int32, (B, H * W), 1)
        mp = jnp.min(jnp.where(ps == mx[:, None], iota_p, NQ), axis=1)
        row = lax.broadcasted_iota(jnp.int32, (B, 1), 0)[:, 0] * (H * W) + mp

        def g(qq, acc):
            gid = qq * QT + lax.broadcasted_iota(jnp.int32, (1, QT), 1)
            ohq = (gid == row[:, None]).astype(jnp.float32)
            eqq = emb_ref[pl.ds(qq * QT, QT), :]
            return acc + _dot(ohq, eqq, (((1,), (0,)), ((), ())))

        mpf_s[...] = lax.fori_loop(
            0, NQT, g, jnp.zeros((B, C), jnp.float32))
        bsc_s[...] = jnp.broadcast_to(mx[:, None], (B, QT))

    @pl.when(s == 0)
    def _():
        mpf = mpf_s[...]
        m2 = jnp.sum(mpf * mpf, axis=1)
        dm = m2[:, None] + b2[None, :] - 2.0 * _dot(
            mpf, chunk, (((1,), (1,)), ((), ())))
        dmp_ref[:, pl.ds(cidx * CT, CT)] = jnp.sqrt(jnp.maximum(dm, 1e-12))
        mc = jnp.min(dm, axis=1)
        am = jnp.argmin(dm, axis=1).astype(jnp.int32)
        prev = jnp.where(i == 0, jnp.inf, runm_ref[:, 0])
        upd = mc < prev
        iota_ct = lax.broadcasted_iota(jnp.int32, (B, CT), 1)
        oh = jnp.where(upd[:, None] & (iota_ct == am[:, None]), 1.0, 0.0)
        cand = _dot(oh, chunk, (((1,), (0,)), ((), ())))
        prev_nns = jnp.where(i == 0, 0.0, nns_ref[...])
        nns_ref[...] = jnp.where(upd[:, None], cand, prev_nns)
        runm_ref[...] = jnp.broadcast_to(
            jnp.where(upd, mc, prev)[:, None], (B, QT))

    @pl.when(s == 1)
    def _():
        nns = nns_ref[...]
        n2 = jnp.sum(nns * nns, axis=1)
        dn = n2[:, None] + b2[None, :] - 2.0 * _dot(
            nns, chunk, (((1,), (1,)), ((), ())))
        dnn_ref[:, pl.ds(cidx * CT, CT)] = jnp.sqrt(jnp.maximum(dn, 1e-12))

    @pl.when(i == 2 * NCT - 1)
    def _():
        iota_c = lax.broadcasted_iota(jnp.int32, (B, NC), 1)
        dd = dnn_ref[...]
        dmp = dmp_ref[...]
        dsup = []
        for _k in range(KNN):
            am = jnp.argmin(dd, axis=1).astype(jnp.int32)
            mask = iota_c == am[:, None]
            dsup.append(jnp.sum(jnp.where(mask, dmp, 0.0), axis=1))
            dd = jnp.where(mask, jnp.inf, dd)
        dsup = jnp.stack(dsup, axis=1)
        mx = jnp.max(dsup, axis=1, keepdims=True)
        e = jnp.exp(dsup - mx)
        wgt = 1.0 - e[:, 0] / jnp.sum(e, axis=1)
        out_ref[...] = (wgt * bsc_s[:, 0])[:, None]


def _score(ps, emb, coreset):
    return pl.pallas_call(
        _score_body,
        grid=(2 * NCT,),
        in_specs=[
            pl.BlockSpec((B, H * W), lambda i: (0, 0)),
            pl.BlockSpec((NQ, C), lambda i: (0, 0)),
            pl.BlockSpec((CT, C), lambda i: (i % NCT, 0)),
        ],
        out_specs=pl.BlockSpec((B, 1), lambda i: (0, 0)),
        out_shape=jax.ShapeDtypeStruct((B, 1), jnp.float32),
        scratch_shapes=[
            pltpu.VMEM((B, C), jnp.float32),
            pltpu.VMEM((B, QT), jnp.float32),
            pltpu.VMEM((B, C), jnp.float32),
            pltpu.VMEM((B, QT), jnp.float32),
            pltpu.VMEM((B, NC), jnp.float32),
            pltpu.VMEM((B, NC), jnp.float32),
        ],
    )(ps, emb, coreset)


def _amap_body(ps_ref, a_ref, o_ref):
    a = a_ref[...]
    p = ps_ref[0]
    t = _dot(a, p, (((1,), (0,)), ((), ())))
    o_ref[0] = _dot(t, a, (((1,), (1,)), ((), ())))


def _amap(ps_img, a_mat):
    return pl.pallas_call(
        _amap_body,
        grid=(B,),
        in_specs=[
            pl.BlockSpec((1, H, W), lambda b: (b, 0, 0)),
            pl.BlockSpec((OUT, H), lambda b: (0, 0)),
        ],
        out_specs=pl.BlockSpec((1, OUT, OUT), lambda b: (b, 0, 0)),
        out_shape=jax.ShapeDtypeStruct((B, OUT, OUT), jnp.float32),
    )(ps_img, a_mat)


def kernel(feat_layer2, feat_layer3, embedding_coreset):
    f2t = jnp.transpose(feat_layer2, (0, 2, 3, 1))
    f3t = jnp.transpose(feat_layer3, (0, 2, 3, 1))
    p2, p3 = _pool(f2t, f3t)
    up3 = jnp.broadcast_to(
        p3[:, :, None, :, None, :],
        (B, H // 2, 2, W // 2, 2, C3)).reshape(B, H, W, C3)
    emb = jnp.concatenate(
        [p2.reshape(NQ, C2), up3.reshape(NQ, C3)], axis=1)
    scores = _knn(emb, embedding_coreset)
    ps = scores.reshape(B, H * W)
    a_score = _score(ps, emb, embedding_coreset).reshape(B)
    amap = _amap(ps.reshape(B, H, W), jnp.asarray(_A_MAP))
    return amap.reshape(B, 1, OUT, OUT), a_score


# speedup vs baseline: 10.9833x; 2.1727x over previous
import numpy as np
import jax
import jax.numpy as jnp
from jax import lax
from jax.experimental import pallas as pl
from jax.experimental.pallas import tpu as pltpu

B, H, W = 8, 28, 28
C2, C3 = 128, 256
C = C2 + C3
NQ = B * H * W
QT = 448
NQT = NQ // QT
QG = 128
NQG = NQ // QG
NC = 16384
CT = 2048
NCT = NC // CT
OUT = 224
KNN = 9

_HIGH = jax.lax.Precision.HIGHEST


def _dot(a, b, dn, prec=_HIGH):
    return lax.dot_general(a, b, dimension_numbers=dn,
                           preferred_element_type=jnp.float32,
                           precision=prec)


_DN_T = (((1,), (1,)), ((), ()))


def _dot3(a, b):
    a_hi = a.astype(jnp.bfloat16)
    a_lo = (a - a_hi.astype(jnp.float32)).astype(jnp.bfloat16)
    b_hi = b.astype(jnp.bfloat16)
    b_lo = (b - b_hi.astype(jnp.float32)).astype(jnp.bfloat16)
    d = jax.lax.Precision.DEFAULT
    hi = _dot(a_hi, b_hi, _DN_T, prec=d)
    m1 = _dot(a_hi, b_lo, _DN_T, prec=d)
    m2 = _dot(a_lo, b_hi, _DN_T, prec=d)
    return hi + m1 + m2


def _map_matrix():
    R = np.zeros((OUT, H), dtype=np.float64)
    scale = H / OUT
    for x in range(OUT):
        pos = (x + 0.5) * scale - 0.5
        w = np.maximum(0.0, 1.0 - np.abs(pos - np.arange(H)))
        R[x] = w / w.sum()
    sigma = 4.0
    ks = 2 * int(4.0 * sigma + 0.5) + 1
    r = ks // 2
    t = np.arange(ks, dtype=np.float64) - r
    g = np.exp(-0.5 * (t / sigma) ** 2)
    g = g / g.sum()
    G = np.zeros((OUT, OUT), dtype=np.float64)
    for x in range(OUT):
        lo = max(0, x - r)
        hi = min(OUT, x + r + 1)
        G[x, lo:hi] = g[lo - x + r:hi - x + r]
    return (G @ R).astype(np.float32)


_A_MAP = _map_matrix()


def _pool_body(x2_ref, x3_ref, o2_ref, o3_ref):
    def pool(x, h, w, c):
        zr = jnp.zeros((1, w, c), jnp.float32)
        xp = jnp.concatenate([zr, x, zr], axis=0)
        zc = jnp.zeros((h + 2, 1, c), jnp.float32)
        xp = jnp.concatenate([zc, xp, zc], axis=1)
        acc = jnp.zeros((h, w, c), jnp.float32)
        for i in range(3):
            for j in range(3):
                acc = acc + xp[i:i + h, j:j + w, :]
        return acc * (1.0 / 9.0)

    o2_ref[0] = pool(x2_ref[0], H, W, C2)
    o3_ref[0] = pool(x3_ref[0], H // 2, W // 2, C3)


def _pool(f2t, f3t):
    return pl.pallas_call(
        _pool_body,
        grid=(B,),
        in_specs=[
            pl.BlockSpec((1, H, W, C2), lambda b: (b, 0, 0, 0)),
            pl.BlockSpec((1, H // 2, W // 2, C3), lambda b: (b, 0, 0, 0)),
        ],
        out_specs=[
            pl.BlockSpec((1, H, W, C2), lambda b: (b, 0, 0, 0)),
            pl.BlockSpec((1, H // 2, W // 2, C3), lambda b: (b, 0, 0, 0)),
        ],
        out_shape=[
            jax.ShapeDtypeStruct((B, H, W, C2), jnp.float32),
            jax.ShapeDtypeStruct((B, H // 2, W // 2, C3), jnp.float32),
        ],
    )(f2t, f3t)


def _knn_body(emb_ref, cs_ref, score_ref, minv, chi, clo, b2s):
    c = pl.program_id(0)
    q = pl.program_id(1)

    @pl.when(q == 0)
    def _():
        ch = cs_ref[...]
        b2s[...] = jnp.broadcast_to(
            jnp.sum(ch * ch, axis=1)[None, :], (8, CT))
        cn = ch * (-2.0)
        hi = cn.astype(jnp.bfloat16)
        chi[...] = hi
        clo[...] = (cn - hi.astype(jnp.float32)).astype(jnp.bfloat16)

    dflt = jax.lax.Precision.DEFAULT
    eq = emb_ref[...]
    a2 = jnp.sum(eq * eq, axis=1)
    e_hi = eq.astype(jnp.bfloat16)
    c_hi = chi[...]
    c_lo = clo[...]
    ab = (_dot(e_hi, c_hi, _DN_T, prec=dflt)
          + _dot(e_hi, c_lo, _DN_T, prec=dflt))
    m = jnp.min(b2s[0:1, :] + ab, axis=1)
    prev_v = jnp.where(c == 0, jnp.inf, minv[q, 0, :])
    new_v = jnp.minimum(m, prev_v)

    @pl.when(c == NCT - 1)
    def _():
        score_ref[0, 0, :] = jnp.sqrt(jnp.maximum(a2 + new_v, 1e-12))

    @pl.when(c < NCT - 1)
    def _():
        minv[q, 0, :] = new_v


def _knn(emb, coreset):
    return pl.pallas_call(
        _knn_body,
        grid=(NCT, NQT),
        in_specs=[
            pl.BlockSpec((QT, C), lambda c, q: (q, 0)),
            pl.BlockSpec((CT, C), lambda c, q: (c, 0)),
        ],
        out_specs=pl.BlockSpec((1, 1, QT), lambda c, q: (q, 0, 0)),
        out_shape=jax.ShapeDtypeStruct((NQT, 1, QT), jnp.float32),
        scratch_shapes=[
            pltpu.VMEM((NQT, 1, QT), jnp.float32),
            pltpu.VMEM((CT, C), jnp.bfloat16),
            pltpu.VMEM((CT, C), jnp.bfloat16),
            pltpu.VMEM((8, CT), jnp.float32),
        ],
    )(emb, coreset)


def _score_body(ps_ref, emb_ref, cs_ref, out_ref, mpf_s, bsc_s, nns_ref,
                runm_ref, dnn_ref, dmp_ref):
    i = pl.program_id(0)
    s = i // NCT
    cidx = i % NCT
    chunk = cs_ref[...]
    b2 = jnp.sum(chunk * chunk, axis=1)

    @pl.when(i == 0)
    def _():
        ps = ps_ref[...]
        mx = jnp.max(ps, axis=1)
        iota_p = lax.broadcasted_iota(jnp.int32, (B, H * W), 1)
        mp = jnp.min(jnp.where(ps == mx[:, None], iota_p, NQ), axis=1)
        row = lax.broadcasted_iota(jnp.int32, (B, 1), 0)[:, 0] * (H * W) + mp

        def g(qq, acc):
            gid = qq * QG + lax.broadcasted_iota(jnp.int32, (1, QG), 1)
            ohq = (gid == row[:, None]).astype(jnp.float32)
            eqq = emb_ref[pl.ds(qq * QG, QG), :]
            return acc + _dot(ohq, eqq, (((1,), (0,)), ((), ())))

        mpf_s[...] = lax.fori_loop(
            0, NQG, g, jnp.zeros((B, C), jnp.float32))
        bsc_s[...] = jnp.broadcast_to(mx[:, None], (B, QG))

    @pl.when(s == 0)
    def _():
        mpf = mpf_s[...]
        m2 = jnp.sum(mpf * mpf, axis=1)
        dm = m2[:, None] + b2[None, :] - 2.0 * _dot(
            mpf, chunk, (((1,), (1,)), ((), ())))
        dmp_ref[:, pl.ds(cidx * CT, CT)] = jnp.sqrt(jnp.maximum(dm, 1e-12))
        mc = jnp.min(dm, axis=1)
        am = jnp.argmin(dm, axis=1).astype(jnp.int32)
        prev = jnp.where(i == 0, jnp.inf, runm_ref[:, 0])
        upd = mc < prev
        iota_ct = lax.broadcasted_iota(jnp.int32, (B, CT), 1)
        oh = jnp.where(upd[:, None] & (iota_ct == am[:, None]), 1.0, 0.0)
        cand = _dot(oh, chunk, (((1,), (0,)), ((), ())))
        prev_nns = jnp.where(i == 0, 0.0, nns_ref[...])
        nns_ref[...] = jnp.where(upd[:, None], cand, prev_nns)
        runm_ref[...] = jnp.broadcast_to(
            jnp.where(upd, mc, prev)[:, None], (B, QG))

    @pl.when(s == 1)
    def _():
        nns = nns_ref[...]
        n2 = jnp.sum(nns * nns, axis=1)
        dn = n2[:, None] + b2[None, :] - 2.0 * _dot(
            nns, chunk, (((1,), (1,)), ((), ())))
        dnn_ref[:, pl.ds(cidx * CT, CT)] = jnp.sqrt(jnp.maximum(dn, 1e-12))

    @pl.when(i == 2 * NCT - 1)
    def _():
        iota_c = lax.broadcasted_iota(jnp.int32, (B, NC), 1)
        dd = dnn_ref[...]
        dmp = dmp_ref[...]
        dsup = []
        for _k in range(KNN):
            am = jnp.argmin(dd, axis=1).astype(jnp.int32)
            mask = iota_c == am[:, None]
            dsup.append(jnp.sum(jnp.where(mask, dmp, 0.0), axis=1))
            dd = jnp.where(mask, jnp.inf, dd)
        dsup = jnp.stack(dsup, axis=1)
        mx = jnp.max(dsup, axis=1, keepdims=True)
        e = jnp.exp(dsup - mx)
        wgt = 1.0 - e[:, 0] / jnp.sum(e, axis=1)
        out_ref[...] = (wgt * bsc_s[:, 0])[:, None]


def _score(ps, emb, coreset):
    return pl.pallas_call(
        _score_body,
        grid=(2 * NCT,),
        in_specs=[
            pl.BlockSpec((B, H * W), lambda i: (0, 0)),
            pl.BlockSpec((NQ, C), lambda i: (0, 0)),
            pl.BlockSpec((CT, C), lambda i: (i % NCT, 0)),
        ],
        out_specs=pl.BlockSpec((B, 1), lambda i: (0, 0)),
        out_shape=jax.ShapeDtypeStruct((B, 1), jnp.float32),
        scratch_shapes=[
            pltpu.VMEM((B, C), jnp.float32),
            pltpu.VMEM((B, QG), jnp.float32),
            pltpu.VMEM((B, C), jnp.float32),
            pltpu.VMEM((B, QG), jnp.float32),
            pltpu.VMEM((B, NC), jnp.float32),
            pltpu.VMEM((B, NC), jnp.float32),
        ],
    )(ps, emb, coreset)


def _amap_body(ps_ref, a_ref, o_ref):
    a = a_ref[...]
    p = ps_ref[0]
    t = _dot(a, p, (((1,), (0,)), ((), ())))
    o_ref[0] = _dot(t, a, (((1,), (1,)), ((), ())))


def _amap(ps_img, a_mat):
    return pl.pallas_call(
        _amap_body,
        grid=(B,),
        in_specs=[
            pl.BlockSpec((1, H, W), lambda b: (b, 0, 0)),
            pl.BlockSpec((OUT, H), lambda b: (0, 0)),
        ],
        out_specs=pl.BlockSpec((1, OUT, OUT), lambda b: (b, 0, 0)),
        out_shape=jax.ShapeDtypeStruct((B, OUT, OUT), jnp.float32),
    )(ps_img, a_mat)


def kernel(feat_layer2, feat_layer3, embedding_coreset):
    f2t = jnp.transpose(feat_layer2, (0, 2, 3, 1))
    f3t = jnp.transpose(feat_layer3, (0, 2, 3, 1))
    p2, p3 = _pool(f2t, f3t)
    up3 = jnp.broadcast_to(
        p3[:, :, None, :, None, :],
        (B, H // 2, 2, W // 2, 2, C3)).reshape(B, H, W, C3)
    emb = jnp.concatenate(
        [p2.reshape(NQ, C2), up3.reshape(NQ, C3)], axis=1)
    scores = _knn(emb, embedding_coreset)
    ps = scores.reshape(B, H * W)
    a_score = _score(ps, emb, embedding_coreset).reshape(B)
    amap = _amap(ps.reshape(B, H, W), jnp.asarray(_A_MAP))
    return amap.reshape(B, 1, OUT, OUT), a_score
